# Initial kernel scaffold; baseline (speedup 1.0000x reference)
import jax
import jax.numpy as jnp
from jax.experimental import pallas as pl


def kernel(x, edge_index, W0l, W0r, b0, W1l, W1r, b1, Wlin, blin):
    raise NotImplementedError("write your pallas kernel here")


# speedup vs baseline: 8.5713x; 8.5713x over previous
import functools

import jax
import jax.numpy as jnp
from jax import lax
from jax.experimental import pallas as pl
from jax.experimental.pallas import tpu as pltpu
from jax.experimental.pallas import tpu_sc as plsc

N = 10000
D = 128
E = 320000

NC = 2
NS = 16
C = 128
T = 79
E_PAD = NC * NS * T * C
N_PAD = 10240
RPT = N_PAD // NS
DR = N_PAD // 128

_f32 = jnp.float32


def _sc_agg_body(compute_deg, x_hbm, src_hbm, dst_hbm, z128_hbm,
                 zdeg_hbm, agg_out, deg_out, src_v, dst_v, rows_v, deg_v,
                 zer_v, acc_sh, sem):
    c = lax.axis_index("c")
    s = lax.axis_index("s")

    pltpu.sync_copy(src_hbm.at[c, s], src_v)
    pltpu.sync_copy(dst_hbm.at[c, s], dst_v)
    pltpu.sync_copy(z128_hbm, zer_v)
    if compute_deg:
        pltpu.sync_copy(zdeg_hbm, deg_v)

    def zero_blk(k, carry):
        pltpu.sync_copy(zer_v, acc_sh.at[pl.ds(s * RPT + k * 8, 8)])
        return carry

    lax.fori_loop(0, RPT // 8, zero_blk, 0)
    plsc.subcore_barrier()

    ones16 = jnp.ones((16,), _f32)

    def chunk(j, carry):
        pltpu.async_copy(x_hbm.at[src_v.at[j]], rows_v, sem).wait()
        pltpu.sync_copy(rows_v, acc_sh.at[dst_v.at[j]], add=True)
        if compute_deg:
            for k in range(C // 16):
                idx16 = dst_v[j, pl.ds(k * 16, 16)]
                plsc.addupdate_scatter(
                    deg_v,
                    [jnp.right_shift(idx16, 7),
                     jnp.bitwise_and(idx16, 127)],
                    ones16)
        return carry

    lax.fori_loop(0, T, chunk, 0)
    plsc.subcore_barrier()

    if compute_deg:
        pltpu.sync_copy(deg_v, deg_out.at[c, s])

    pltpu.sync_copy(acc_sh.at[pl.ds(s * RPT, RPT)],
                    agg_out.at[c, pl.ds(s * RPT, RPT)])


def _make_sc_agg(compute_deg):
    mesh = plsc.VectorSubcoreMesh(core_axis_name="c", subcore_axis_name="s",
                                  num_cores=NC, num_subcores=NS)
    out_type = [jax.ShapeDtypeStruct((NC, N_PAD, D), _f32)]
    if compute_deg:
        out_type.append(jax.ShapeDtypeStruct((NC, NS, DR, 128), _f32))
    scratch = [
        pltpu.VMEM((T, C), jnp.int32),
        pltpu.VMEM((T, C), jnp.int32),
        pltpu.VMEM((C, D), _f32),
        pltpu.VMEM((DR, 128), _f32),
        pltpu.VMEM((8, 128), _f32),
        pltpu.VMEM_SHARED((N_PAD, D), _f32),
        pltpu.SemaphoreType.DMA,
    ]
    body = functools.partial(_sc_agg_body, compute_deg)
    if not compute_deg:
        def body(x_hbm, src_hbm, dst_hbm, z128_hbm, agg_out, src_v, dst_v,
                 rows_v, zer_v, acc_sh, sem):
            _sc_agg_body(False, x_hbm, src_hbm, dst_hbm, z128_hbm, None,
                         agg_out, None, src_v, dst_v, rows_v, None, zer_v,
                         acc_sh, sem)
        scratch = scratch[:3] + scratch[4:]
    return pl.kernel(
        body,
        out_type=out_type,
        mesh=mesh,
        scratch_types=scratch,
        compiler_params=pltpu.CompilerParams(needs_layout_passes=False),
        name="sc_segsum" + ("_deg" if compute_deg else ""),
    )


BN = 1024


def _tc0_body(x_ref, p0_ref, p1_ref, dp_ref, wl_ref, wr_ref, b_ref,
              h_ref):
    deg = jnp.maximum(jnp.sum(dp_ref[...], axis=0), 1.0)
    mean = (p0_ref[...] + p1_ref[...]) / deg[:, None]
    h = (jnp.dot(x_ref[...], wl_ref[...], preferred_element_type=_f32)
         + jnp.dot(mean, wr_ref[...], preferred_element_type=_f32)
         + b_ref[...])
    h_ref[...] = jnp.maximum(h, 0.0)


def _tc1_body(h_ref, p0_ref, p1_ref, dp_ref, wl_ref, wr_ref, b_ref,
              wo_ref, bo_ref, o_ref):
    deg = jnp.maximum(jnp.sum(dp_ref[...], axis=0), 1.0)
    mean = (p0_ref[...] + p1_ref[...]) / deg[:, None]
    t = (jnp.dot(h_ref[...], wl_ref[...], preferred_element_type=_f32)
         + jnp.dot(mean, wr_ref[...], preferred_element_type=_f32)
         + b_ref[...])
    t = jnp.maximum(t, 0.0)
    o_ref[...] = (jnp.dot(t, wo_ref[...], preferred_element_type=_f32)
                  + bo_ref[...])


def _row_spec():
    return pl.BlockSpec((BN, D), lambda i: (i, 0))


def _deg_spec():
    return pl.BlockSpec((NC * NS, BN), lambda i: (0, i))


def _w_spec():
    return pl.BlockSpec((D, D), lambda i: (0, 0))


def _b_spec():
    return pl.BlockSpec((1, D), lambda i: (0, 0))


def _tc_layer0(x, p0, p1, dp, Wl, Wr, b):
    return pl.pallas_call(
        _tc0_body,
        grid=(N_PAD // BN,),
        in_specs=[_row_spec(), _row_spec(), _row_spec(), _deg_spec(),
                  _w_spec(), _w_spec(), _b_spec()],
        out_specs=_row_spec(),
        out_shape=jax.ShapeDtypeStruct((N_PAD, D), _f32),
    )(x, p0, p1, dp, Wl, Wr, b)


def _tc_layer1(h, p0, p1, dp, Wl, Wr, b, Wo, bo):
    return pl.pallas_call(
        _tc1_body,
        grid=(N_PAD // BN,),
        in_specs=[_row_spec(), _row_spec(), _row_spec(), _deg_spec(),
                  _w_spec(), _w_spec(), _b_spec(), _w_spec(), _b_spec()],
        out_specs=_row_spec(),
        out_shape=jax.ShapeDtypeStruct((N_PAD, D), _f32),
    )(h, p0, p1, dp, Wl, Wr, b, Wo, bo)


def kernel(x, edge_index, W0l, W0r, b0, W1l, W1r, b1, Wlin, blin):
    x_pad = jnp.pad(x, ((0, N_PAD - N), (0, 0)))
    src = edge_index[0]
    dst = edge_index[1]
    npad = E_PAD - E
    pad_ar = jnp.arange(npad, dtype=jnp.int32)
    src_p = jnp.concatenate([src, pad_ar % N]).reshape(NC, NS, T, C)
    dst_p = jnp.concatenate([dst, N + pad_ar % (N_PAD - N)]).reshape(
        NC, NS, T, C)
    z128 = jnp.zeros((8, 128), _f32)
    zdeg = jnp.zeros((DR, 128), _f32)
    b0r = b0.reshape(1, D)
    b1r = b1.reshape(1, D)
    blinr = blin.reshape(1, D)

    agg0, deg = _make_sc_agg(True)(x_pad, src_p, dst_p, z128, zdeg)
    degp = deg.reshape(NC * NS, N_PAD)
    h = _tc_layer0(x_pad, agg0[0], agg0[1], degp, W0l, W0r, b0r)

    agg1 = _make_sc_agg(False)(h, src_p, dst_p, z128)[0]
    out = _tc_layer1(h, agg1[0], agg1[1], degp, W1l, W1r, b1r, Wlin, blinr)

    return out[:N]


# speedup vs baseline: 13.1245x; 1.5312x over previous
import functools

import jax
import jax.numpy as jnp
from jax import lax
from jax.experimental import pallas as pl
from jax.experimental.pallas import tpu as pltpu
from jax.experimental.pallas import tpu_sc as plsc

N = 10000
D = 128
E = 320000

NC = 2
NS = 16
C = 128
T = 160
NBUF = 4
E_PAD = NS * T * C
N_PAD = 10240
RPT = N_PAD // NS
DR = N_PAD // 128
DH = D // NC

_f32 = jnp.float32


def _sc_agg_body(compute_deg, x2_hbm, src_hbm, dst_hbm, z_hbm, zdeg_hbm,
                 agg_out, deg_out, src_v, dst_v, r0, r1, r2, r3, deg_v,
                 zer_v, acc_sh, gsem, ssem):
    c = lax.axis_index("c")
    s = lax.axis_index("s")
    bufs = [r0, r1, r2, r3]

    pltpu.sync_copy(src_hbm.at[c, s], src_v)
    pltpu.sync_copy(dst_hbm.at[c, s], dst_v)
    pltpu.sync_copy(z_hbm, zer_v)
    if compute_deg:
        pltpu.sync_copy(zdeg_hbm, deg_v)

    def zero_blk(k, carry):
        pltpu.sync_copy(zer_v, acc_sh.at[pl.ds(s * RPT + k * 8, 8)])
        return carry

    lax.fori_loop(0, RPT // 8, zero_blk, 0)
    plsc.subcore_barrier()

    ones16 = jnp.ones((16,), _f32)

    for b in range(NBUF):
        pltpu.async_copy(x2_hbm.at[src_v.at[b]], bufs[b], gsem)

    def quad(q, carry):
        for b in range(NBUF):
            j = q * NBUF + b
            pltpu.make_async_copy(
                x2_hbm.at[src_v.at[j]], bufs[b], gsem).wait()
            sdesc = pltpu.async_copy(
                bufs[b], acc_sh.at[dst_v.at[j]], ssem, add=True)
            if compute_deg:
                @pl.when(c == 0)
                def _():
                    for k in range(C // 16):
                        idx16 = dst_v[j, pl.ds(k * 16, 16)]
                        plsc.addupdate_scatter(
                            deg_v,
                            [jnp.right_shift(idx16, 7),
                             jnp.bitwise_and(idx16, 127)],
                            ones16)
            sdesc.wait()
            nxt = j + NBUF

            @pl.when(nxt < T)
            def _():
                pltpu.async_copy(x2_hbm.at[src_v.at[nxt]], bufs[b], gsem)
        return carry

    lax.fori_loop(0, T // NBUF, quad, 0)
    plsc.subcore_barrier()

    if compute_deg:
        @pl.when(c == 0)
        def _():
            pltpu.sync_copy(deg_v, deg_out.at[s])

    pltpu.sync_copy(acc_sh.at[pl.ds(s * RPT, RPT)],
                    agg_out.at[c, pl.ds(s * RPT, RPT)])


def _make_sc_agg(compute_deg):
    mesh = plsc.VectorSubcoreMesh(core_axis_name="c", subcore_axis_name="s",
                                  num_cores=NC, num_subcores=NS)
    out_type = [jax.ShapeDtypeStruct((NC, N_PAD, DH), _f32)]
    if compute_deg:
        out_type.append(jax.ShapeDtypeStruct((NS, DR, 128), _f32))
    scratch = [
        pltpu.VMEM((T, C), jnp.int32),
        pltpu.VMEM((T, C), jnp.int32),
        pltpu.VMEM((C, DH), _f32),
        pltpu.VMEM((C, DH), _f32),
        pltpu.VMEM((C, DH), _f32),
        pltpu.VMEM((C, DH), _f32),
        pltpu.VMEM((DR, 128), _f32),
        pltpu.VMEM((8, DH), _f32),
        pltpu.VMEM_SHARED((N_PAD, DH), _f32),
        pltpu.SemaphoreType.DMA,
        pltpu.SemaphoreType.DMA,
    ]
    body = functools.partial(_sc_agg_body, compute_deg)
    if not compute_deg:
        def body(x2_hbm, src_hbm, dst_hbm, z_hbm, agg_out, src_v, dst_v,
                 r0, r1, r2, r3, zer_v, acc_sh, gsem, ssem):
            _sc_agg_body(False, x2_hbm, src_hbm, dst_hbm, z_hbm, None,
                         agg_out, None, src_v, dst_v, r0, r1, r2, r3, None,
                         zer_v, acc_sh, gsem, ssem)
        scratch = scratch[:6] + scratch[7:]
    return pl.kernel(
        body,
        out_type=out_type,
        mesh=mesh,
        scratch_types=scratch,
        compiler_params=pltpu.CompilerParams(
            needs_layout_passes=False, use_tc_tiling_on_sc=False),
        name="sc_segsum" + ("_deg" if compute_deg else ""),
    )


BN = 1024


def _tc0_body(x_ref, p0_ref, p1_ref, dp_ref, wl_ref, wr_ref, b_ref,
              h_ref):
    deg = jnp.maximum(jnp.sum(dp_ref[...], axis=0), 1.0)
    agg = jnp.concatenate([p0_ref[...], p1_ref[...]], axis=1)
    mean = agg / deg[:, None]
    h = (jnp.dot(x_ref[...], wl_ref[...], preferred_element_type=_f32)
         + jnp.dot(mean, wr_ref[...], preferred_element_type=_f32)
         + b_ref[...])
    h_ref[...] = jnp.maximum(h, 0.0)


def _tc1_body(h_ref, p0_ref, p1_ref, dp_ref, wl_ref, wr_ref, b_ref,
              wo_ref, bo_ref, o_ref):
    deg = jnp.maximum(jnp.sum(dp_ref[...], axis=0), 1.0)
    agg = jnp.concatenate([p0_ref[...], p1_ref[...]], axis=1)
    mean = agg / deg[:, None]
    t = (jnp.dot(h_ref[...], wl_ref[...], preferred_element_type=_f32)
         + jnp.dot(mean, wr_ref[...], preferred_element_type=_f32)
         + b_ref[...])
    t = jnp.maximum(t, 0.0)
    o_ref[...] = (jnp.dot(t, wo_ref[...], preferred_element_type=_f32)
                  + bo_ref[...])


def _row_spec():
    return pl.BlockSpec((BN, D), lambda i: (i, 0))


def _half_spec():
    return pl.BlockSpec((BN, DH), lambda i: (i, 0))


def _deg_spec():
    return pl.BlockSpec((NS, BN), lambda i: (0, i))


def _w_spec():
    return pl.BlockSpec((D, D), lambda i: (0, 0))


def _b_spec():
    return pl.BlockSpec((1, D), lambda i: (0, 0))


def _tc_layer0(x, p0, p1, dp, Wl, Wr, b):
    return pl.pallas_call(
        _tc0_body,
        grid=(N_PAD // BN,),
        in_specs=[_row_spec(), _half_spec(), _half_spec(), _deg_spec(),
                  _w_spec(), _w_spec(), _b_spec()],
        out_specs=_row_spec(),
        out_shape=jax.ShapeDtypeStruct((N_PAD, D), _f32),
    )(x, p0, p1, dp, Wl, Wr, b)


def _tc_layer1(h, p0, p1, dp, Wl, Wr, b, Wo, bo):
    return pl.pallas_call(
        _tc1_body,
        grid=(N_PAD // BN,),
        in_specs=[_row_spec(), _half_spec(), _half_spec(), _deg_spec(),
                  _w_spec(), _w_spec(), _b_spec(), _w_spec(), _b_spec()],
        out_specs=_row_spec(),
        out_shape=jax.ShapeDtypeStruct((N_PAD, D), _f32),
    )(h, p0, p1, dp, Wl, Wr, b, Wo, bo)


def kernel(x, edge_index, W0l, W0r, b0, W1l, W1r, b1, Wlin, blin):
    x_pad = jnp.pad(x, ((0, N_PAD - N), (0, 0)))
    src = edge_index[0]
    dst = edge_index[1]
    npad = E_PAD - E
    pad_ar = jnp.arange(npad, dtype=jnp.int32)
    src_t = jnp.concatenate([src, pad_ar % N]).reshape(NS, T, C)
    dst_t = jnp.concatenate([dst, N + pad_ar % (N_PAD - N)]).reshape(
        NS, T, C)
    core_ofs = jnp.arange(NC, dtype=jnp.int32).reshape(NC, 1, 1, 1)
    src_p = 2 * src_t[None] + core_ofs
    dst_p = jnp.broadcast_to(dst_t[None], (NC, NS, T, C))
    z = jnp.zeros((8, DH), _f32)
    zdeg = jnp.zeros((DR, 128), _f32)
    b0r = b0.reshape(1, D)
    b1r = b1.reshape(1, D)
    blinr = blin.reshape(1, D)

    x2 = x_pad.reshape(2 * N_PAD, DH)
    agg0, deg = _make_sc_agg(True)(x2, src_p, dst_p, z, zdeg)
    degp = deg.reshape(NS, N_PAD)
    h = _tc_layer0(x_pad, agg0[0], agg0[1], degp, W0l, W0r, b0r)

    h2 = h.reshape(2 * N_PAD, DH)
    agg1 = _make_sc_agg(False)(h2, src_p, dst_p, z)[0]
    out = _tc_layer1(h, agg1[0], agg1[1], degp, W1l, W1r, b1r, Wlin, blinr)

    return out[:N]
